# baseline (speedup 1.0000x reference)
import jax
import jax.numpy as jnp
from jax.experimental import pallas as pl
from jax.experimental.pallas import tpu as pltpu

_NDIM = 3
_EPS = 1e-5
_LANE = 128


def _leaky(x):
    return jnp.where(x >= 0, x, 0.01 * x)


def _bn(x, g, b):
    n = x.shape[0]
    xr = x.reshape(n // _LANE, _LANE, x.shape[1])
    s1 = jnp.sum(xr, axis=(0, 1)) * (1.0 / n)
    s2 = jnp.sum(xr * xr, axis=(0, 1)) * (1.0 / n)
    var = s2 - s1 * s1
    scale = g * jax.lax.rsqrt(var + _EPS)
    return x * scale + (b - s1 * scale)


def _mm(x, w, transposed=False):
    xb = x.astype(jnp.bfloat16)
    xe = (x - xb.astype(jnp.float32)).astype(jnp.bfloat16)
    wb = w.astype(jnp.bfloat16)
    if transposed:
        dn = (((1,), (1,)), ((), ()))

        def d(a):
            return jax.lax.dot_general(a, wb, dn,
                                       preferred_element_type=jnp.float32)
    else:
        def d(a):
            return jnp.dot(a, wb, preferred_element_type=jnp.float32)
    return d(xb) + d(xe)


_LAYOUT = (
    ("riW0", 16), ("riB0", 1), ("riG0", 1), ("riBt0", 1),
    ("riW1_t", 64), ("riB1", 1), ("riG1", 1), ("riBt1", 1),
    ("d0W0", 64), ("d0B0", 1), ("d0G0", 1), ("d0Bt0", 1),
    ("d0W1p_t", 8), ("d0W1w_t", 64), ("d0B1p", 1), ("d0B1w", 1),
    ("d1W0", 64), ("d1B0", 1), ("d1G0", 1), ("d1Bt0", 1),
    ("d1W1p_t", 8), ("d1W1w_t", 64), ("d1B1p", 1), ("d1B1w", 1),
    ("roW0", 64), ("roB0", 1), ("roG0", 1), ("roBt0", 1),
    ("roW1_t", 32), ("roB1", 1),
)
_OFFSETS = {}
_r = 0
for _name, _rows in _LAYOUT:
    _OFFSETS[_name] = (_r, _rows)
    _r += _rows
_PACK_ROWS = _r


def _forward_kernel(w_hbm, pos_hbm, prm_hbm, pos_out_hbm, w_out_hbm,
                    w_buf, pos_buf, prm, op_buf, ow_buf, in_sems, out_sems):
    copies = [
        pltpu.make_async_copy(w_hbm, w_buf, in_sems.at[0]),
        pltpu.make_async_copy(pos_hbm, pos_buf, in_sems.at[1]),
        pltpu.make_async_copy(prm_hbm, prm, in_sems.at[2]),
    ]
    for c in copies:
        c.start()
    for c in copies:
        c.wait()

    def mat(name):
        o, r = _OFFSETS[name]
        return prm[o:o + r, :]

    def vec(name, width=_LANE):
        o, _ = _OFFSETS[name]
        return prm[o:o + 1, :width]

    x = _leaky(_bn(_mm(w_buf[...][:, :16], mat("riW0")) + vec("riB0"),
                   vec("riG0"), vec("riBt0")))
    w = _leaky(_bn(_mm(x, mat("riW1_t"), True) + vec("riB1", 64),
                   vec("riG1", 64), vec("riBt1", 64)))

    dp = jnp.zeros((x.shape[0], _NDIM), jnp.float32)
    for p in ("d0", "d1"):
        h = _leaky(_bn(_mm(w, mat(p + "W0")) + vec(p + "B0"),
                       vec(p + "G0"), vec(p + "Bt0")))
        dp = dp + _mm(h, mat(p + "W1p_t")[:_NDIM, :], True) + vec(p + "B1p", _NDIM)
        w = w + _mm(h, mat(p + "W1w_t"), True) + vec(p + "B1w", 64)

    op_buf[...] = pos_buf[...] + dp
    pos_copy = pltpu.make_async_copy(op_buf, pos_out_hbm, out_sems.at[0])
    pos_copy.start()

    h = _leaky(_bn(_mm(w, mat("roW0")) + vec("roB0"),
                   vec("roG0"), vec("roBt0")))
    ow_buf[...] = _mm(h, mat("roW1_t"), True) + vec("roB1", 32)
    w_copy = pltpu.make_async_copy(ow_buf, w_out_hbm, out_sems.at[1])
    w_copy.start()

    pos_copy.wait()
    w_copy.wait()


def _pack_params(params):
    pieces = []

    def put(a, rows):
        if a.ndim == 1:
            a = a.reshape(1, -1)
        r, c = a.shape
        if c < _LANE:
            a = jnp.pad(a, ((0, 0), (0, _LANE - c)))
        if r < rows:
            a = jnp.pad(a, ((0, rows - r), (0, 0)))
        pieces.append(a)

    ri0, ri1 = params["readin"]
    put(ri0["W"], 16), put(ri0["b"], 1), put(ri0["gamma"], 1), put(ri0["beta"], 1)
    put(ri1["W"].T, 64), put(ri1["b"], 1), put(ri1["gamma"], 1), put(ri1["beta"], 1)
    for blk in params["blocks"]:
        l0, l1 = blk["delta"]
        put(l0["W"], 64), put(l0["b"], 1), put(l0["gamma"], 1), put(l0["beta"], 1)
        put(l1["W"][:, :_NDIM].T, 8)
        put(l1["W"][:, _NDIM:].T, 64)
        put(l1["b"][:_NDIM], 1)
        put(l1["b"][_NDIM:], 1)
    ro0, ro1 = params["readout"]
    put(ro0["W"], 64), put(ro0["b"], 1), put(ro0["gamma"], 1), put(ro0["beta"], 1)
    put(ro1["W"].T, 32), put(ro1["b"], 1)
    return jnp.concatenate(pieces, axis=0)


def kernel(positions, weights, params, batch):
    del batch
    n = positions.shape[0]
    packed = _pack_params(params)
    out_dim = params["readout"][1]["W"].shape[1]

    hbm = pl.BlockSpec(memory_space=pltpu.MemorySpace.HBM)
    pos_out, w_out = pl.pallas_call(
        _forward_kernel,
        in_specs=[hbm, hbm, hbm],
        out_specs=(hbm, hbm),
        out_shape=(
            jax.ShapeDtypeStruct((n, _NDIM), jnp.float32),
            jax.ShapeDtypeStruct((n, out_dim), jnp.float32),
        ),
        scratch_shapes=(
            pltpu.VMEM(weights.shape, jnp.float32),
            pltpu.VMEM((n, _NDIM), jnp.float32),
            pltpu.VMEM((_PACK_ROWS, _LANE), jnp.float32),
            pltpu.VMEM((n, _NDIM), jnp.float32),
            pltpu.VMEM((n, out_dim), jnp.float32),
            pltpu.SemaphoreType.DMA((3,)),
            pltpu.SemaphoreType.DMA((2,)),
        ),
    )(weights, positions, packed)
    return pos_out, w_out


# speedup vs baseline: 1.7183x; 1.7183x over previous
import jax
import jax.numpy as jnp
from jax.experimental import pallas as pl

_NDIM = 3
_EPS = 1e-5
_LANE = 128

_ROWS = (
    ("riW0", 16), ("riB0", 1), ("riG0", 1), ("riBt0", 1),
    ("riB1", 1), ("riG1", 1), ("riBt1", 1),
    ("d0W0", 64), ("d0B0", 1), ("d0G0", 1), ("d0Bt0", 1),
    ("d0B1p", 1), ("d0B1w", 1),
    ("d1W0", 64), ("d1B0", 1), ("d1G0", 1), ("d1Bt0", 1),
    ("d1B1p", 1), ("d1B1w", 1),
    ("roW0", 64), ("roB0", 1), ("roG0", 1), ("roBt0", 1),
    ("roB1", 1),
)
_OFF = {}
_r = 0
for _name, _nrows in _ROWS:
    _OFF[_name] = (_r, _nrows)
    _r += _nrows
_PACK_ROWS = _r


def _leaky(x):
    return jnp.where(x >= 0, x, 0.01 * x)


def _split(x):
    xb = x.astype(jnp.bfloat16)
    xe = (x - xb.astype(jnp.float32)).astype(jnp.bfloat16)
    return xb, xe


def _mm2(xb, xe, w):
    wb = w.astype(jnp.bfloat16)
    return (jnp.dot(xb, wb, preferred_element_type=jnp.float32)
            + jnp.dot(xe, wb, preferred_element_type=jnp.float32))


def _bn_act(x, g, b):
    n = x.shape[0]
    xr = x.reshape(n // _LANE, _LANE, x.shape[1])
    s1 = jnp.sum(xr, axis=(0, 1)) * (1.0 / n)
    s2 = jnp.sum(xr * xr, axis=(0, 1)) * (1.0 / n)
    var = s2 - s1 * s1
    scale = g * jax.lax.rsqrt(var + _EPS)
    return _leaky(x * scale + (b - s1 * scale))


def _forward_kernel(w_in_ref, pos_ref, prm_ref, riW1_ref, d0W1_ref, d1W1_ref,
                    roW1_ref, pos_out, w_out):
    prm = prm_ref[...]

    def mat(name):
        o, nr = _OFF[name]
        return prm[o:o + nr, :]

    def vec(name, width=_LANE):
        o, _ = _OFF[name]
        return prm[o:o + 1, :width]

    xb, xe = _split(w_in_ref[...])
    x = _bn_act(_mm2(xb, xe, mat("riW0")) + vec("riB0"),
                vec("riG0"), vec("riBt0"))
    xb, xe = _split(x)
    w = _bn_act(_mm2(xb, xe, riW1_ref[...]) + vec("riB1", 64),
                vec("riG1", 64), vec("riBt1", 64))

    dp = jnp.zeros((x.shape[0], _NDIM), jnp.float32)
    for p, w1_ref in (("d0", d0W1_ref), ("d1", d1W1_ref)):
        wb_, we_ = _split(w)
        h = _bn_act(_mm2(wb_, we_, mat(p + "W0")) + vec(p + "B0"),
                    vec(p + "G0"), vec(p + "Bt0"))
        w1 = w1_ref[...]
        hb, he = _split(h)
        dp = dp + _mm2(hb, he, w1[:, :_NDIM]) + vec(p + "B1p", _NDIM)
        w = w + _mm2(hb, he, w1[:, _NDIM:]) + vec(p + "B1w", 64)

    wb_, we_ = _split(w)
    h = _bn_act(_mm2(wb_, we_, mat("roW0")) + vec("roB0"),
                vec("roG0"), vec("roBt0"))
    hb, he = _split(h)
    out_dim = w_out.shape[1]
    w_out[...] = _mm2(hb, he, roW1_ref[...]) + vec("roB1", out_dim)
    pos_out[...] = pos_ref[...] + dp


def _pack_params(params):
    z = jnp.zeros((), jnp.float32)

    def zrow(k):
        return jnp.broadcast_to(z, (k,))

    ri0, ri1 = params["readin"]
    ro0, ro1 = params["readout"]
    pieces = [
        ri0["W"].reshape(-1), ri0["b"], ri0["gamma"], ri0["beta"],
        ri1["b"], zrow(64), ri1["gamma"], zrow(64), ri1["beta"], zrow(64),
    ]
    for blk in params["blocks"]:
        l0, l1 = blk["delta"]
        pieces += [
            l0["W"].reshape(-1), l0["b"], l0["gamma"], l0["beta"],
            l1["b"][:_NDIM], zrow(_LANE - _NDIM),
            l1["b"][_NDIM:], zrow(64),
        ]
    pieces += [
        ro0["W"].reshape(-1), ro0["b"], ro0["gamma"], ro0["beta"],
        ro1["b"], zrow(_LANE - ro1["b"].shape[0]),
    ]
    return jnp.concatenate(pieces).reshape(_PACK_ROWS, _LANE)


def kernel(positions, weights, params, batch):
    del batch
    n = positions.shape[0]
    packed = _pack_params(params)
    ri1 = params["readin"][1]["W"]
    d0W1 = params["blocks"][0]["delta"][1]["W"]
    d1W1 = params["blocks"][1]["delta"][1]["W"]
    roW1 = params["readout"][1]["W"]
    out_dim = roW1.shape[1]

    pos_out, w_out = pl.pallas_call(
        _forward_kernel,
        out_shape=(
            jax.ShapeDtypeStruct((n, _NDIM), jnp.float32),
            jax.ShapeDtypeStruct((n, out_dim), jnp.float32),
        ),
    )(weights, positions, packed, ri1, d0W1, d1W1, roW1)
    return pos_out, w_out


# speedup vs baseline: 1.8833x; 1.0960x over previous
import jax
import jax.numpy as jnp
from jax.experimental import pallas as pl

_NDIM = 3
_EPS = 1e-5
_LANE = 128


def _leaky(x):
    return jnp.where(x >= 0, x, 0.01 * x)


def _split(x):
    xb = x.astype(jnp.bfloat16)
    xe = (x - xb.astype(jnp.float32)).astype(jnp.bfloat16)
    return xb, xe


def _mm2(xb, xe, w):
    wb = w.astype(jnp.bfloat16)
    return (jnp.dot(xb, wb, preferred_element_type=jnp.float32)
            + jnp.dot(xe, wb, preferred_element_type=jnp.float32))


def _bn_act(x, g, b):
    n = x.shape[0]
    xr = x.reshape(n // _LANE, _LANE, x.shape[1])
    s1 = jnp.sum(xr, axis=(0, 1)) * (1.0 / n)
    s2 = jnp.sum(xr * xr, axis=(0, 1)) * (1.0 / n)
    var = s2 - s1 * s1
    scale = g * jax.lax.rsqrt(var + _EPS)
    return _leaky(x * scale + (b - s1 * scale))


def _forward_kernel(w_in_ref, pos_ref, *refs):
    args = refs[:26]
    pos_out, w_out = refs[26], refs[27]

    it = iter(args)

    def take(n):
        return [next(it)[...] for _ in range(n)]

    riW0, riB0, riG0, riBt0, riW1, riB1, riG1, riBt1 = take(8)
    blk0 = take(6)
    blk1 = take(6)
    roW0, roB0, roG0, roBt0, roW1, roB1 = take(6)

    xb, xe = _split(w_in_ref[...])
    x = _bn_act(_mm2(xb, xe, riW0) + riB0, riG0, riBt0)
    xb, xe = _split(x)
    w = _bn_act(_mm2(xb, xe, riW1) + riB1, riG1, riBt1)

    dp = jnp.zeros((x.shape[0], _NDIM), jnp.float32)
    for dW0, dB0, dG0, dBt0, dW1, dB1 in (blk0, blk1):
        wb_, we_ = _split(w)
        h = _bn_act(_mm2(wb_, we_, dW0) + dB0, dG0, dBt0)
        hb, he = _split(h)
        dp = dp + _mm2(hb, he, dW1[:, :_NDIM]) + dB1[:_NDIM]
        w = w + _mm2(hb, he, dW1[:, _NDIM:]) + dB1[_NDIM:]

    wb_, we_ = _split(w)
    h = _bn_act(_mm2(wb_, we_, roW0) + roB0, roG0, roBt0)
    hb, he = _split(h)
    w_out[...] = _mm2(hb, he, roW1) + roB1
    pos_out[...] = pos_ref[...] + dp


def kernel(positions, weights, params, batch):
    del batch
    n = positions.shape[0]

    flat = []
    for p in params["readin"]:
        flat += [p["W"], p["b"], p["gamma"], p["beta"]]
    for blk in params["blocks"]:
        l0, l1 = blk["delta"]
        flat += [l0["W"], l0["b"], l0["gamma"], l0["beta"], l1["W"], l1["b"]]
    ro0, ro1 = params["readout"]
    flat += [ro0["W"], ro0["b"], ro0["gamma"], ro0["beta"], ro1["W"], ro1["b"]]

    out_dim = ro1["W"].shape[1]
    pos_out, w_out = pl.pallas_call(
        _forward_kernel,
        out_shape=(
            jax.ShapeDtypeStruct((n, _NDIM), jnp.float32),
            jax.ShapeDtypeStruct((n, out_dim), jnp.float32),
        ),
    )(weights, positions, *flat)
    return pos_out, w_out


# speedup vs baseline: 2.0552x; 1.0913x over previous
import jax
import jax.numpy as jnp
from jax.experimental import pallas as pl

_NDIM = 3
_EPS = 1e-5


def _leaky(x):
    return jnp.where(x >= 0, x, 0.01 * x)


def _bn(x, g, b):
    mu = jnp.mean(x, axis=0, keepdims=True)
    var = jnp.mean((x - mu) ** 2, axis=0, keepdims=True)
    return g * ((x - mu) * jax.lax.rsqrt(var + _EPS)) + b


def _dense(x, w, b):
    return jnp.dot(x, w, preferred_element_type=jnp.float32) + b


def _forward_kernel(pos_ref, w_ref, *refs):
    args = [r[...] for r in refs[:-2]]
    out_pos, out_w = refs[-2], refs[-1]

    it = iter(args)

    def take(n):
        return [next(it) for _ in range(n)]

    riW0, riB0, riG0, riBt0, riW1, riB1, riG1, riBt1 = take(8)

    x = w_ref[...]
    x = _leaky(_bn(_dense(x, riW0, riB0), riG0, riBt0))
    w = _leaky(_bn(_dense(x, riW1, riB1), riG1, riBt1))
    pos = pos_ref[...]

    for _ in range(2):
        dW0, dB0, dG0, dBt0, dW1, dB1 = take(6)
        h = _leaky(_bn(_dense(w, dW0, dB0), dG0, dBt0))
        pos = pos + _dense(h, dW1[:, :_NDIM], dB1[:_NDIM])
        w = w + _dense(h, dW1[:, _NDIM:], dB1[_NDIM:])

    roW0, roB0, roG0, roBt0, roW1, roB1 = take(6)
    h = _leaky(_bn(_dense(w, roW0, roB0), roG0, roBt0))
    out_pos[...] = pos
    out_w[...] = _dense(h, roW1, roB1)


def kernel(positions, weights, params, batch):
    del batch
    n = positions.shape[0]

    flat = []
    for p in params["readin"]:
        flat += [p["W"], p["b"], p["gamma"], p["beta"]]
    for blk in params["blocks"]:
        l0, l1 = blk["delta"]
        flat += [l0["W"], l0["b"], l0["gamma"], l0["beta"], l1["W"], l1["b"]]
    ro0, ro1 = params["readout"]
    flat += [ro0["W"], ro0["b"], ro0["gamma"], ro0["beta"], ro1["W"], ro1["b"]]

    out_dim = ro1["W"].shape[1]
    pos_out, w_out = pl.pallas_call(
        _forward_kernel,
        out_shape=(
            jax.ShapeDtypeStruct((n, _NDIM), jnp.float32),
            jax.ShapeDtypeStruct((n, out_dim), jnp.float32),
        ),
    )(positions, weights, *flat)
    return pos_out, w_out
